# baseline (device time: 26224 ns/iter reference)
import jax
import jax.numpy as jnp
from jax import lax
from jax.experimental import pallas as pl
from jax.experimental.pallas import tpu as pltpu

T_PER = 256
H = 128
D = 512
F = 1024
E_LOCAL = 2


def _top2_weights(g):
    v1 = jnp.max(g, axis=1, keepdims=True)
    t1 = g >= v1
    gm = jnp.where(t1, -jnp.inf, g)
    v2 = jnp.max(gm, axis=1, keepdims=True)
    t2 = gm >= v2
    w1 = 1.0 / (1.0 + jnp.exp(v2 - v1))
    return jnp.where(t1, w1, 0.0) + jnp.where(t2, 1.0 - w1, 0.0)


def kernel(x, router, W1, W2):
    def body(x_ref, r_ref, w1_any, w2_any, out_ref,
             w1_ref, w2_ref, xs_ref, xr_ref, rr_ref, ws_ref, wr_ref,
             ps_ref, pr_ref, fs_ref, fr_ref,
             send_sems, recv_sems, local_sems):
        ix = lax.axis_index("x")
        iy = lax.axis_index("y")
        nbr_y = (ix, 1 - iy)
        nbr_x = (1 - ix, iy)

        cp_w1 = pltpu.make_async_copy(w1_any, w1_ref, local_sems.at[0])
        cp_w1.start()
        cp_w2 = pltpu.make_async_copy(w2_any, w2_ref, local_sems.at[1])
        cp_w2.start()

        barrier_sem = pltpu.get_barrier_semaphore()
        for nbr in (nbr_y, nbr_x):
            pl.semaphore_signal(barrier_sem, inc=1, device_id=nbr,
                                device_id_type=pl.DeviceIdType.MESH)
        pl.semaphore_wait(barrier_sem, 2)

        def rdma(i, src, dst, dev):
            return pltpu.make_async_remote_copy(
                src_ref=src, dst_ref=dst,
                send_sem=send_sems.at[i], recv_sem=recv_sems.at[i],
                device_id=dev, device_id_type=pl.DeviceIdType.MESH)

        rows = pl.ds(ix * H, H)

        xs_ref[...] = x_ref[rows, :].astype(jnp.bfloat16)
        rdma_x = rdma(0, xs_ref, xr_ref, nbr_y)
        rdma_x.start()
        rdma_r = rdma(1, r_ref, rr_ref, nbr_y)
        rdma_r.start()

        cp_w1.wait()
        cp_w2.wait()
        w1b = [w1_ref[le].astype(jnp.bfloat16) for le in range(E_LOCAL)]
        w2b = [w2_ref[le].astype(jnp.bfloat16) for le in range(E_LOCAL)]

        def expert(xb, le):
            h = jnp.dot(xb, w1b[le], preferred_element_type=jnp.float32)
            h = jnp.maximum(h, 0.0).astype(jnp.bfloat16)
            return jnp.dot(h, w2b[le], preferred_element_type=jnp.float32)

        xbm = xs_ref[...]
        o_my = [expert(xbm, le) for le in range(E_LOCAL)]

        rdma_r.wait()
        x_half = x_ref[rows, :]
        g = jnp.concatenate(
            [jnp.dot(x_half, r_ref[...], preferred_element_type=jnp.float32),
             jnp.dot(x_half, rr_ref[...], preferred_element_type=jnp.float32)],
            axis=1)
        wt = _top2_weights(g)

        ws_ref[...] = wt[:, E_LOCAL:]
        rdma_w = rdma(2, ws_ref, wr_ref, nbr_y)
        rdma_w.start()

        acc_my = o_my[0] * wt[:, 0:1] + o_my[1] * wt[:, 1:2]

        rdma_x.wait()
        rdma_w.wait()
        xbn = xr_ref[...]
        wr = wr_ref[...]
        acc_nb = (expert(xbn, 0) * wr[:, 0:1]
                  + expert(xbn, 1) * wr[:, 1:2])

        ps_ref[...] = acc_nb.astype(jnp.bfloat16)
        rdma_p = rdma(3, ps_ref, pr_ref, nbr_y)
        rdma_p.start()
        rdma_p.wait()

        final = acc_my + pr_ref[...].astype(jnp.float32)
        out_ref[rows, :] = final

        fs_ref[...] = final.astype(jnp.bfloat16)
        rdma_f = rdma(4, fs_ref, fr_ref, nbr_x)
        rdma_f.start()
        rdma_f.wait()
        out_ref[pl.ds((1 - ix) * H, H), :] = fr_ref[...].astype(jnp.float32)

    return pl.pallas_call(
        body,
        out_shape=jax.ShapeDtypeStruct((T_PER, D), jnp.float32),
        in_specs=[
            pl.BlockSpec(memory_space=pltpu.VMEM),
            pl.BlockSpec(memory_space=pltpu.VMEM),
            pl.BlockSpec(memory_space=pl.ANY),
            pl.BlockSpec(memory_space=pl.ANY),
        ],
        out_specs=pl.BlockSpec(memory_space=pltpu.VMEM),
        scratch_shapes=[
            pltpu.VMEM((E_LOCAL, D, F), jnp.float32),
            pltpu.VMEM((E_LOCAL, F, D), jnp.float32),
            pltpu.VMEM((H, D), jnp.bfloat16),
            pltpu.VMEM((H, D), jnp.bfloat16),
            pltpu.VMEM((D, E_LOCAL), jnp.float32),
            pltpu.VMEM((H, E_LOCAL), jnp.float32),
            pltpu.VMEM((H, E_LOCAL), jnp.float32),
            pltpu.VMEM((H, D), jnp.bfloat16),
            pltpu.VMEM((H, D), jnp.bfloat16),
            pltpu.VMEM((H, D), jnp.bfloat16),
            pltpu.VMEM((H, D), jnp.bfloat16),
            pltpu.SemaphoreType.DMA((5,)),
            pltpu.SemaphoreType.DMA((5,)),
            pltpu.SemaphoreType.DMA((2,)),
        ],
        compiler_params=pltpu.CompilerParams(collective_id=0),
    )(x, router, W1, W2)


# device time: 24327 ns/iter; 1.0780x vs baseline; 1.0780x over previous
import jax
import jax.numpy as jnp
from jax import lax
from jax.experimental import pallas as pl
from jax.experimental.pallas import tpu as pltpu

T_PER = 256
H = 128
D = 512
F = 1024
E_LOCAL = 2


def _top2_weights(g):
    v1 = jnp.max(g, axis=1, keepdims=True)
    t1 = g >= v1
    gm = jnp.where(t1, -jnp.inf, g)
    v2 = jnp.max(gm, axis=1, keepdims=True)
    t2 = gm >= v2
    w1 = 1.0 / (1.0 + jnp.exp(v2 - v1))
    return jnp.where(t1, w1, 0.0) + jnp.where(t2, 1.0 - w1, 0.0)


def kernel(x, router, W1, W2):
    def body(x_any, r_ref, w1_any, w2_any, out_any,
             w1_ref, w2_ref, xh_ref, xs_ref, xr_ref, rr_ref, ws_ref, wr_ref,
             ps_ref, pr_ref, fs_ref, fr_ref, fv_ref, fw_ref,
             send_sems, recv_sems, local_sems):
        ix = lax.axis_index("x")
        iy = lax.axis_index("y")
        nbr_y = (ix, 1 - iy)
        nbr_x = (1 - ix, iy)
        rows = pl.ds(ix * H, H)
        other_rows = pl.ds((1 - ix) * H, H)

        cp_w1 = pltpu.make_async_copy(w1_any, w1_ref, local_sems.at[0])
        cp_w1.start()
        cp_w2 = pltpu.make_async_copy(w2_any, w2_ref, local_sems.at[1])
        cp_w2.start()
        cp_x = pltpu.make_async_copy(x_any.at[rows, :], xh_ref,
                                     local_sems.at[2])
        cp_x.start()

        barrier_sem = pltpu.get_barrier_semaphore()
        for nbr in (nbr_y, nbr_x):
            pl.semaphore_signal(barrier_sem, inc=1, device_id=nbr,
                                device_id_type=pl.DeviceIdType.MESH)
        pl.semaphore_wait(barrier_sem, 2)

        def rdma(i, src, dst, dev):
            return pltpu.make_async_remote_copy(
                src_ref=src, dst_ref=dst,
                send_sem=send_sems.at[i], recv_sem=recv_sems.at[i],
                device_id=dev, device_id_type=pl.DeviceIdType.MESH)

        rdma_r = rdma(1, r_ref, rr_ref, nbr_y)
        rdma_r.start()

        cp_x.wait()
        xs_ref[...] = xh_ref[...].astype(jnp.bfloat16)
        rdma_x = rdma(0, xs_ref, xr_ref, nbr_y)
        rdma_x.start()

        rdma_r.wait()
        x_half = xh_ref[...]
        g = jnp.concatenate(
            [jnp.dot(x_half, r_ref[...], preferred_element_type=jnp.float32),
             jnp.dot(x_half, rr_ref[...], preferred_element_type=jnp.float32)],
            axis=1)
        wt = _top2_weights(g)

        ws_ref[...] = wt[:, E_LOCAL:]
        rdma_w = rdma(2, ws_ref, wr_ref, nbr_y)
        rdma_w.start()

        cp_w1.wait()
        cp_w2.wait()
        w1b = [w1_ref[le].astype(jnp.bfloat16) for le in range(E_LOCAL)]
        w2b = [w2_ref[le].astype(jnp.bfloat16) for le in range(E_LOCAL)]

        def expert(xb, le):
            h = jnp.dot(xb, w1b[le], preferred_element_type=jnp.float32)
            h = jnp.maximum(h, 0.0).astype(jnp.bfloat16)
            return jnp.dot(h, w2b[le], preferred_element_type=jnp.float32)

        xbm = xs_ref[...]
        acc_my = (expert(xbm, 0) * wt[:, 0:1]
                  + expert(xbm, 1) * wt[:, 1:2])

        rdma_x.wait()
        xbn = xr_ref[...]
        o_nb = [expert(xbn, le) for le in range(E_LOCAL)]
        rdma_w.wait()
        wr = wr_ref[...]
        acc_nb = o_nb[0] * wr[:, 0:1] + o_nb[1] * wr[:, 1:2]

        ps_ref[...] = acc_nb.astype(jnp.bfloat16)
        rdma_p = rdma(3, ps_ref, pr_ref, nbr_y)
        rdma_p.start()
        rdma_p.wait()

        final = acc_my + pr_ref[...].astype(jnp.float32)
        fv_ref[...] = final
        cp_o1 = pltpu.make_async_copy(fv_ref, out_any.at[rows, :],
                                      local_sems.at[3])
        cp_o1.start()

        fs_ref[...] = final.astype(jnp.bfloat16)
        rdma_f = rdma(4, fs_ref, fr_ref, nbr_x)
        rdma_f.start()
        rdma_f.wait()
        fw_ref[...] = fr_ref[...].astype(jnp.float32)
        cp_o2 = pltpu.make_async_copy(fw_ref, out_any.at[other_rows, :],
                                      local_sems.at[4])
        cp_o2.start()
        cp_o1.wait()
        cp_o2.wait()

    return pl.pallas_call(
        body,
        out_shape=jax.ShapeDtypeStruct((T_PER, D), jnp.float32),
        in_specs=[
            pl.BlockSpec(memory_space=pl.ANY),
            pl.BlockSpec(memory_space=pltpu.VMEM),
            pl.BlockSpec(memory_space=pl.ANY),
            pl.BlockSpec(memory_space=pl.ANY),
        ],
        out_specs=pl.BlockSpec(memory_space=pl.ANY),
        scratch_shapes=[
            pltpu.VMEM((E_LOCAL, D, F), jnp.float32),
            pltpu.VMEM((E_LOCAL, F, D), jnp.float32),
            pltpu.VMEM((H, D), jnp.float32),
            pltpu.VMEM((H, D), jnp.bfloat16),
            pltpu.VMEM((H, D), jnp.bfloat16),
            pltpu.VMEM((D, E_LOCAL), jnp.float32),
            pltpu.VMEM((H, E_LOCAL), jnp.float32),
            pltpu.VMEM((H, E_LOCAL), jnp.float32),
            pltpu.VMEM((H, D), jnp.bfloat16),
            pltpu.VMEM((H, D), jnp.bfloat16),
            pltpu.VMEM((H, D), jnp.bfloat16),
            pltpu.VMEM((H, D), jnp.bfloat16),
            pltpu.VMEM((H, D), jnp.float32),
            pltpu.VMEM((H, D), jnp.float32),
            pltpu.SemaphoreType.DMA((5,)),
            pltpu.SemaphoreType.DMA((5,)),
            pltpu.SemaphoreType.DMA((5,)),
        ],
        compiler_params=pltpu.CompilerParams(collective_id=0),
    )(x, router, W1, W2)


# device time: 18259 ns/iter; 1.4362x vs baseline; 1.3323x over previous
import jax
import jax.numpy as jnp
from jax import lax
from jax.experimental import pallas as pl
from jax.experimental.pallas import tpu as pltpu

T_PER = 256
H = 128
D = 512
F = 1024
E_LOCAL = 2


def _top2_weights(g):
    v1 = jnp.max(g, axis=1, keepdims=True)
    t1 = g >= v1
    gm = jnp.where(t1, -jnp.inf, g)
    v2 = jnp.max(gm, axis=1, keepdims=True)
    t2 = gm >= v2
    w1 = 1.0 / (1.0 + jnp.exp(v2 - v1))
    return jnp.where(t1, w1, 0.0) + jnp.where(t2, 1.0 - w1, 0.0)


def kernel(x, router, W1, W2):
    def body(x_any, r_ref, w1_any, w2_any, out_any,
             w1_ref, w2_ref, xh_ref, xs_ref, xr_ref, rr_ref, ws_ref, wr_ref,
             ps_ref, pr_ref, fs_ref, fr_ref, fv_ref, fw_ref,
             send_sems, recv_sems, local_sems):
        ix = lax.axis_index("x")
        iy = lax.axis_index("y")
        nbr_y = (ix, 1 - iy)
        nbr_x = (1 - ix, iy)
        rows = pl.ds(ix * H, H)
        other_rows = pl.ds((1 - ix) * H, H)

        cp_w1 = pltpu.make_async_copy(w1_any, w1_ref, local_sems.at[0])
        cp_w1.start()
        cp_w2 = pltpu.make_async_copy(w2_any, w2_ref, local_sems.at[1])
        cp_w2.start()
        cp_x = pltpu.make_async_copy(x_any.at[rows, :], xh_ref,
                                     local_sems.at[2])
        cp_x.start()

        barrier_sem = pltpu.get_barrier_semaphore()
        for nbr in (nbr_y, nbr_x):
            pl.semaphore_signal(barrier_sem, inc=1, device_id=nbr,
                                device_id_type=pl.DeviceIdType.MESH)
        pl.semaphore_wait(barrier_sem, 2)

        def rdma(i, src, dst, dev):
            return pltpu.make_async_remote_copy(
                src_ref=src, dst_ref=dst,
                send_sem=send_sems.at[i], recv_sem=recv_sems.at[i],
                device_id=dev, device_id_type=pl.DeviceIdType.MESH)

        rdma_r = rdma(1, r_ref, rr_ref, nbr_y)
        rdma_r.start()

        cp_x.wait()
        xs_ref[...] = xh_ref[...].astype(jnp.bfloat16)
        rdma_x = rdma(0, xs_ref, xr_ref, nbr_y)
        rdma_x.start()

        rdma_r.wait()
        x_half = xh_ref[...]
        g = jnp.concatenate(
            [jnp.dot(x_half, r_ref[...], preferred_element_type=jnp.float32),
             jnp.dot(x_half, rr_ref[...], preferred_element_type=jnp.float32)],
            axis=1)
        wt = _top2_weights(g)

        ws_ref[...] = wt[:, E_LOCAL:]
        rdma_w = rdma(2, ws_ref, wr_ref, nbr_y)
        rdma_w.start()

        cp_w1.wait()
        cp_w2.wait()
        w1b = [w1_ref[le].astype(jnp.bfloat16) for le in range(E_LOCAL)]
        w2b = [w2_ref[le].astype(jnp.bfloat16) for le in range(E_LOCAL)]

        def expert(xb, le):
            h = jnp.dot(xb, w1b[le], preferred_element_type=jnp.float32)
            h = jnp.maximum(h, 0.0).astype(jnp.bfloat16)
            return jnp.dot(h, w2b[le], preferred_element_type=jnp.float32)

        xbm = xs_ref[...]
        acc_my = (expert(xbm, 0) * wt[:, 0:1]
                  + expert(xbm, 1) * wt[:, 1:2])

        rdma_x.wait()
        xbn = xr_ref[...]
        o_nb = [expert(xbn, le) for le in range(E_LOCAL)]
        rdma_w.wait()
        wr = wr_ref[...]
        acc_nb = o_nb[0] * wr[:, 0:1] + o_nb[1] * wr[:, 1:2]

        ps_ref[...] = acc_nb.astype(jnp.bfloat16)
        rdma_p = rdma(3, ps_ref, pr_ref, nbr_y)
        rdma_p.start()
        rdma_p.wait()

        final = acc_my + pr_ref[...].astype(jnp.float32)
        fv_ref[...] = final
        cp_o1 = pltpu.make_async_copy(fv_ref, out_any.at[rows, :],
                                      local_sems.at[3])
        cp_o1.start()

        fs_ref[...] = final.astype(jnp.bfloat16)
        rdma_f = rdma(4, fs_ref, fr_ref, nbr_x)
        rdma_f.start()
        rdma_f.wait()
        fw_ref[...] = fr_ref[...].astype(jnp.float32)
        cp_o2 = pltpu.make_async_copy(fw_ref, out_any.at[other_rows, :],
                                      local_sems.at[4])
        cp_o2.start()
        cp_o1.wait()
        cp_o2.wait()

    return pl.pallas_call(
        body,
        out_shape=jax.ShapeDtypeStruct((T_PER, D), jnp.float32),
        in_specs=[
            pl.BlockSpec(memory_space=pltpu.MemorySpace.HBM),
            pl.BlockSpec(memory_space=pltpu.VMEM),
            pl.BlockSpec(memory_space=pltpu.MemorySpace.HBM),
            pl.BlockSpec(memory_space=pltpu.MemorySpace.HBM),
        ],
        out_specs=pl.BlockSpec(memory_space=pltpu.MemorySpace.HBM),
        scratch_shapes=[
            pltpu.VMEM((E_LOCAL, D, F), jnp.float32),
            pltpu.VMEM((E_LOCAL, F, D), jnp.float32),
            pltpu.VMEM((H, D), jnp.float32),
            pltpu.VMEM((H, D), jnp.bfloat16),
            pltpu.VMEM((H, D), jnp.bfloat16),
            pltpu.VMEM((D, E_LOCAL), jnp.float32),
            pltpu.VMEM((H, E_LOCAL), jnp.float32),
            pltpu.VMEM((H, E_LOCAL), jnp.float32),
            pltpu.VMEM((H, D), jnp.bfloat16),
            pltpu.VMEM((H, D), jnp.bfloat16),
            pltpu.VMEM((H, D), jnp.bfloat16),
            pltpu.VMEM((H, D), jnp.bfloat16),
            pltpu.VMEM((H, D), jnp.float32),
            pltpu.VMEM((H, D), jnp.float32),
            pltpu.SemaphoreType.DMA((5,)),
            pltpu.SemaphoreType.DMA((5,)),
            pltpu.SemaphoreType.DMA((5,)),
        ],
        compiler_params=pltpu.CompilerParams(collective_id=0),
    )(pltpu.with_memory_space_constraint(x, pltpu.MemorySpace.HBM),
      router,
      pltpu.with_memory_space_constraint(W1, pltpu.MemorySpace.HBM),
      pltpu.with_memory_space_constraint(W2, pltpu.MemorySpace.HBM))


# device time: 15960 ns/iter; 1.6431x vs baseline; 1.1440x over previous
import jax
import jax.numpy as jnp
from jax import lax
from jax.experimental import pallas as pl
from jax.experimental.pallas import tpu as pltpu

T_PER = 256
H = 128
D = 512
F = 1024
E_LOCAL = 2


def _top2_weights(g):
    v1 = jnp.max(g, axis=1, keepdims=True)
    t1 = g >= v1
    gm = jnp.where(t1, -jnp.inf, g)
    v2 = jnp.max(gm, axis=1, keepdims=True)
    t2 = gm >= v2
    w1 = 1.0 / (1.0 + jnp.exp(v2 - v1))
    return jnp.where(t1, w1, 0.0) + jnp.where(t2, 1.0 - w1, 0.0)


def kernel(x, router, W1, W2):
    def body(x_any, r_ref, w1_any, w2_any, out_any,
             w1_ref, w2_ref, xh_ref, xs_ref, xr_ref, rr_ref, ws_ref, wr_ref,
             es_ref, er_ref, ps_ref, pr_ref, qr_ref, fv_ref, fw_ref,
             send_sems, recv_sems, local_sems):
        ix = lax.axis_index("x")
        iy = lax.axis_index("y")
        nbr_y = (ix, 1 - iy)
        nbr_x = (1 - ix, iy)
        diag = (1 - ix, 1 - iy)
        rows = pl.ds(ix * H, H)
        other_rows = pl.ds((1 - ix) * H, H)

        cp_w1 = pltpu.make_async_copy(w1_any, w1_ref, local_sems.at[0])
        cp_w1.start()
        cp_w2 = pltpu.make_async_copy(w2_any, w2_ref, local_sems.at[1])
        cp_w2.start()
        cp_x = pltpu.make_async_copy(x_any.at[rows, :], xh_ref,
                                     local_sems.at[2])
        cp_x.start()

        barrier_sem = pltpu.get_barrier_semaphore()
        for nbr in (nbr_y, nbr_x, diag):
            pl.semaphore_signal(barrier_sem, inc=1, device_id=nbr,
                                device_id_type=pl.DeviceIdType.MESH)
        pl.semaphore_wait(barrier_sem, 3)

        def rdma(i, src, dst, dev):
            return pltpu.make_async_remote_copy(
                src_ref=src, dst_ref=dst,
                send_sem=send_sems.at[i], recv_sem=recv_sems.at[i],
                device_id=dev, device_id_type=pl.DeviceIdType.MESH)

        rdma_r = rdma(1, r_ref, rr_ref, nbr_y)
        rdma_r.start()

        cp_x.wait()
        xs_ref[...] = xh_ref[...].astype(jnp.bfloat16)
        rdma_x = rdma(0, xs_ref, xr_ref, nbr_y)
        rdma_x.start()

        rdma_r.wait()
        x_half = xh_ref[...]
        g = jnp.concatenate(
            [jnp.dot(x_half, r_ref[...], preferred_element_type=jnp.float32),
             jnp.dot(x_half, rr_ref[...], preferred_element_type=jnp.float32)],
            axis=1)
        wt = _top2_weights(g)

        ws_ref[...] = wt[:, E_LOCAL:]
        rdma_w = rdma(2, ws_ref, wr_ref, nbr_y)
        rdma_w.start()

        cp_w1.wait()
        cp_w2.wait()
        w1b = [w1_ref[le].astype(jnp.bfloat16) for le in range(E_LOCAL)]
        w2b = [w2_ref[le].astype(jnp.bfloat16) for le in range(E_LOCAL)]

        def expert(xb, le):
            h = jnp.dot(xb, w1b[le], preferred_element_type=jnp.float32)
            h = jnp.maximum(h, 0.0).astype(jnp.bfloat16)
            return jnp.dot(h, w2b[le], preferred_element_type=jnp.float32)

        xbm = xs_ref[...]
        acc_my = (expert(xbm, 0) * wt[:, 0:1]
                  + expert(xbm, 1) * wt[:, 1:2])

        es_ref[...] = acc_my.astype(jnp.bfloat16)
        rdma_e = rdma(3, es_ref, er_ref, nbr_x)
        rdma_e.start()

        rdma_x.wait()
        xbn = xr_ref[...]
        o_nb = [expert(xbn, le) for le in range(E_LOCAL)]
        rdma_w.wait()
        wr = wr_ref[...]
        acc_nb = o_nb[0] * wr[:, 0:1] + o_nb[1] * wr[:, 1:2]

        ps_ref[...] = acc_nb.astype(jnp.bfloat16)
        rdma_p2 = rdma(5, ps_ref, qr_ref, diag)
        rdma_p2.start()
        rdma_p = rdma(4, ps_ref, pr_ref, nbr_y)
        rdma_p.start()

        rdma_p.wait_recv()
        fv_ref[...] = acc_my + pr_ref[...].astype(jnp.float32)
        cp_o1 = pltpu.make_async_copy(fv_ref, out_any.at[rows, :],
                                      local_sems.at[3])
        cp_o1.start()

        rdma_e.wait()
        rdma_p2.wait_recv()
        fw_ref[...] = (er_ref[...].astype(jnp.float32)
                       + qr_ref[...].astype(jnp.float32))
        cp_o2 = pltpu.make_async_copy(fw_ref, out_any.at[other_rows, :],
                                      local_sems.at[4])
        cp_o2.start()

        rdma_p.wait_send()
        rdma_p2.wait_send()
        cp_o1.wait()
        cp_o2.wait()

    return pl.pallas_call(
        body,
        out_shape=jax.ShapeDtypeStruct((T_PER, D), jnp.float32),
        in_specs=[
            pl.BlockSpec(memory_space=pltpu.MemorySpace.HBM),
            pl.BlockSpec(memory_space=pltpu.VMEM),
            pl.BlockSpec(memory_space=pltpu.MemorySpace.HBM),
            pl.BlockSpec(memory_space=pltpu.MemorySpace.HBM),
        ],
        out_specs=pl.BlockSpec(memory_space=pltpu.MemorySpace.HBM),
        scratch_shapes=[
            pltpu.VMEM((E_LOCAL, D, F), jnp.float32),
            pltpu.VMEM((E_LOCAL, F, D), jnp.float32),
            pltpu.VMEM((H, D), jnp.float32),
            pltpu.VMEM((H, D), jnp.bfloat16),
            pltpu.VMEM((H, D), jnp.bfloat16),
            pltpu.VMEM((D, E_LOCAL), jnp.float32),
            pltpu.VMEM((H, E_LOCAL), jnp.float32),
            pltpu.VMEM((H, E_LOCAL), jnp.float32),
            pltpu.VMEM((H, D), jnp.bfloat16),
            pltpu.VMEM((H, D), jnp.bfloat16),
            pltpu.VMEM((H, D), jnp.bfloat16),
            pltpu.VMEM((H, D), jnp.bfloat16),
            pltpu.VMEM((H, D), jnp.bfloat16),
            pltpu.VMEM((H, D), jnp.float32),
            pltpu.VMEM((H, D), jnp.float32),
            pltpu.SemaphoreType.DMA((6,)),
            pltpu.SemaphoreType.DMA((6,)),
            pltpu.SemaphoreType.DMA((5,)),
        ],
        compiler_params=pltpu.CompilerParams(collective_id=0),
    )(pltpu.with_memory_space_constraint(x, pltpu.MemorySpace.HBM),
      router,
      pltpu.with_memory_space_constraint(W1, pltpu.MemorySpace.HBM),
      pltpu.with_memory_space_constraint(W2, pltpu.MemorySpace.HBM))
